# Initial kernel scaffold; baseline (speedup 1.0000x reference)
import jax
import jax.numpy as jnp
from jax.experimental import pallas as pl


def kernel(g):
    raise NotImplementedError("write your pallas kernel here")


# speedup vs baseline: 72.8300x; 72.8300x over previous
import functools

import jax
import jax.numpy as jnp
from jax import lax
from jax.experimental import pallas as pl
from jax.experimental.pallas import tpu as pltpu

_K_FRAC = 0.5


def _tc_body(k, g_ref, out_ref):
    g = g_ref[...]
    u = lax.bitcast_convert_type(g, jnp.uint32)
    mono = jnp.where(u >> 31 == jnp.uint32(1), ~u, u | jnp.uint32(0x80000000))
    s = lax.bitcast_convert_type(mono ^ jnp.uint32(0x80000000), jnp.int32)

    B = g.shape[0]

    def step(i, v):
        b = 31 - i
        cand = v | (jnp.uint32(1) << b)
        t = lax.bitcast_convert_type(cand ^ jnp.uint32(0x80000000), jnp.int32)
        cnt = jnp.sum((s < t).astype(jnp.int32), axis=1, keepdims=True)
        return jnp.where(cnt < k, cand, v)

    v = lax.fori_loop(0, 32, step, jnp.zeros((B, 1), jnp.uint32))
    t = lax.bitcast_convert_type(v ^ jnp.uint32(0x80000000), jnp.int32)
    out_ref[...] = jnp.where(s <= t, jnp.float32(0.0), g)


@jax.jit
def kernel(g):
    B, N = g.shape
    k = int(N * _K_FRAC)
    return pl.pallas_call(
        functools.partial(_tc_body, k),
        out_shape=jax.ShapeDtypeStruct((B, N), g.dtype),
    )(g)
